# baseline (speedup 1.0000x reference)
import functools

import jax
import jax.numpy as jnp
from jax import lax
from jax.experimental import pallas as pl
from jax.experimental.pallas import tpu as pltpu
from jax.experimental.pallas import tpu_sc as plsc

N = 10000
D = 128
NS_SUP = 2
E_TOT = 2 * 320000
NC = 2
NSC = 16
NW = NC * NSC
CHUNK = 128
EB = 8
NCH = 160
NB = NCH // EB
E_PAD = NW * CHUNK * NCH
SHARE = 624
SHARE_SPLIT = (128, 128, 128, 128, 112)
REM_START = NSC * SHARE
REM = N - REM_START


def _mm_body(x_ref, w_ref, o_ref):
    o_ref[...] = jnp.dot(x_ref[...], w_ref[0],
                         preferred_element_type=jnp.float32)[None]


def _tc_matmul(x, W):
    BR = 2000
    out = pl.pallas_call(
        _mm_body,
        grid=(NS_SUP, N // BR),
        in_specs=[
            pl.BlockSpec((BR, D), lambda s, i: (i, 0)),
            pl.BlockSpec((1, D, D), lambda s, i: (s, 0, 0)),
        ],
        out_specs=pl.BlockSpec((1, BR, D), lambda s, i: (s, i, 0)),
        out_shape=jax.ShapeDtypeStruct((NS_SUP, N, D), jnp.float32),
    )(x, W)
    return out.reshape(NS_SUP * N, D)


def _add_body(p_ref, o_ref):
    o_ref[...] = p_ref[0] + p_ref[1]


def _tc_combine(partial):
    BR = 2000
    return pl.pallas_call(
        _add_body,
        grid=(N // BR,),
        in_specs=[pl.BlockSpec((NC, BR, D), lambda i: (0, i, 0))],
        out_specs=pl.BlockSpec((BR, D), lambda i: (i, 0)),
        out_shape=jax.ShapeDtypeStruct((N, D), jnp.float32),
    )(partial)


def _splat_lane(vec, lane):
    idx = jnp.full((16, 1), lane, jnp.int32)
    return lax.gather(
        vec, idx,
        lax.GatherDimensionNumbers(
            offset_dims=(), collapsed_slice_dims=(0,), start_index_map=(0,)),
        slice_sizes=(1,),
        mode=lax.GatherScatterMode.PROMISE_IN_BOUNDS)


_sc_mesh = plsc.VectorSubcoreMesh(
    core_axis_name="c", subcore_axis_name="s", num_cores=NC, num_subcores=NSC
)


@functools.partial(
    pl.kernel,
    out_type=jax.ShapeDtypeStruct((NC, N, D), jnp.float32),
    mesh=_sc_mesh,
    scratch_types=[
        pltpu.VMEM((2, EB, CHUNK), jnp.int32),
        pltpu.VMEM((2, EB, CHUNK), jnp.int32),
        pltpu.VMEM((2, EB, CHUNK), jnp.float32),
        pltpu.VMEM((CHUNK, D), jnp.float32),
        pltpu.VMEM((CHUNK, D), jnp.float32),
        pltpu.VMEM_SHARED((N, D), jnp.float32),
        pltpu.SemaphoreType.DMA,
        pltpu.SemaphoreType.DMA,
        pltpu.SemaphoreType.DMA,
        pltpu.SemaphoreType.DMA,
        pltpu.SemaphoreType.DMA,
        pltpu.SemaphoreType.DMA,
        pltpu.SemaphoreType.DMA,
    ],
)
def _sc_scatter(xw_hbm, src_hbm, dst_hbm, ew_hbm, out_hbm,
                src_v, dst_v, ew_v, rows0_v, rows1_v, acc,
                gsem_a0, gsem_a1, gsem_b0, gsem_b1, esem, ssem_0, ssem_1):
    rows_v = rows0_v
    H = CHUNK // 2

    def _gather_start(es, j, rows, s0, s1):
        pltpu.async_copy(xw_hbm.at[src_v.at[es, j, pl.ds(0, H)]],
                         rows.at[pl.ds(0, H)], s0)
        pltpu.async_copy(xw_hbm.at[src_v.at[es, j, pl.ds(H, H)]],
                         rows.at[pl.ds(H, H)], s1)

    def _scatter(es, j, rows, s0, s1):
        pltpu.async_copy(rows.at[pl.ds(0, H)],
                         acc.at[dst_v.at[es, j, pl.ds(0, H)]], s0, add=True)
        pltpu.async_copy(rows.at[pl.ds(H, H)],
                         acc.at[dst_v.at[es, j, pl.ds(H, H)]], s1, add=True)
        pltpu.make_async_copy(
            rows.at[pl.ds(0, H)],
            acc.at[dst_v.at[es, j, pl.ds(0, H)]], s0).wait()
        pltpu.make_async_copy(
            rows.at[pl.ds(H, H)],
            acc.at[dst_v.at[es, j, pl.ds(H, H)]], s1).wait()

    def _gather_wait(es, j, rows, s0, s1):
        pltpu.make_async_copy(xw_hbm.at[src_v.at[es, j, pl.ds(0, H)]],
                              rows.at[pl.ds(0, H)], s0).wait()
        pltpu.make_async_copy(xw_hbm.at[src_v.at[es, j, pl.ds(H, H)]],
                              rows.at[pl.ds(H, H)], s1).wait()
    cid = lax.axis_index("c")
    sid = lax.axis_index("s")
    wid = cid * NSC + sid

    def _zero_body(i, carry):
        z = jnp.zeros((16,), jnp.float32)
        for g in range(8):
            rows_v[i, pl.ds(g * 16, 16)] = z
        return carry

    lax.fori_loop(0, CHUNK, _zero_body, 0)
    off = 0
    for ln in SHARE_SPLIT:
        pltpu.sync_copy(rows_v.at[pl.ds(0, ln)],
                        acc.at[pl.ds(sid * SHARE + off, ln)])
        off += ln

    @pl.when(sid == 0)
    def _zero_rem():
        pltpu.sync_copy(rows_v.at[pl.ds(0, REM)], acc.at[pl.ds(REM_START, REM)])

    plsc.subcore_barrier()

    def _scale(rows, es, j):
        def _group_body(gr, c2):
            wv = ew_v[es, j, pl.ds(gr * 16, 16)]
            for ln in range(16):
                w = _splat_lane(wv, ln)
                e = gr * 16 + ln
                for g in range(8):
                    rows[e, pl.ds(g * 16, 16)] = rows[e, pl.ds(g * 16, 16)] * w
            return c2

        lax.fori_loop(0, CHUNK // 16, _group_body, 0)

    def _stage_start(b, slot):
        bsl = pl.ds(b * EB, EB)
        pltpu.async_copy(src_hbm.at[wid, bsl], src_v.at[slot], esem)
        pltpu.async_copy(dst_hbm.at[wid, bsl], dst_v.at[slot], esem)
        pltpu.async_copy(ew_hbm.at[wid, bsl], ew_v.at[slot], esem)

    def _stage_wait(b, slot):
        bsl = pl.ds(b * EB, EB)
        pltpu.make_async_copy(src_hbm.at[wid, bsl], src_v.at[slot], esem).wait()
        pltpu.make_async_copy(dst_hbm.at[wid, bsl], dst_v.at[slot], esem).wait()
        pltpu.make_async_copy(ew_hbm.at[wid, bsl], ew_v.at[slot], esem).wait()

    _stage_start(0, 0)
    _stage_wait(0, 0)

    def _block_body(b, carry):
        es = b & 1
        @pl.when(b < NB - 1)
        def _stage_next():
            _stage_start(b + 1, 1 - es)

        _gather_start(es, 0, rows0_v, gsem_a0, gsem_a1)

        def _pair_body(p, c1):
            ja = 2 * p
            jb = ja + 1
            _gather_wait(es, ja, rows0_v, gsem_a0, gsem_a1)
            _gather_start(es, jb, rows1_v, gsem_b0, gsem_b1)
            _scale(rows0_v, es, ja)
            _scatter(es, ja, rows0_v, ssem_0, ssem_1)

            _gather_wait(es, jb, rows1_v, gsem_b0, gsem_b1)

            @pl.when(p < EB // 2 - 1)
            def _prefetch_next():
                _gather_start(es, ja + 2, rows0_v, gsem_a0, gsem_a1)

            _scale(rows1_v, es, jb)
            _scatter(es, jb, rows1_v, ssem_0, ssem_1)
            return c1

        lax.fori_loop(0, EB // 2, _pair_body, 0)

        @pl.when(b < NB - 1)
        def _stage_drain():
            _stage_wait(b + 1, 1 - es)

        return carry

    lax.fori_loop(0, NB, _block_body, 0)
    plsc.subcore_barrier()

    off = 0
    for ln in SHARE_SPLIT:
        sl = pl.ds(sid * SHARE + off, ln)
        pltpu.sync_copy(acc.at[sl], out_hbm.at[cid, sl])
        off += ln

    @pl.when(sid == 0)
    def _write_rem():
        sl = pl.ds(REM_START, REM)
        pltpu.sync_copy(acc.at[sl], out_hbm.at[cid, sl])


def kernel(x, edge_index_0, edge_weight_0, edge_index_1, edge_weight_1, W):
    xw = _tc_matmul(x, W)

    src = jnp.concatenate([
        edge_index_0[1].astype(jnp.int32),
        edge_index_1[1].astype(jnp.int32) + N,
    ])
    dst = jnp.concatenate([
        edge_index_0[0].astype(jnp.int32),
        edge_index_1[0].astype(jnp.int32),
    ])
    ew = jnp.concatenate([edge_weight_0, edge_weight_1])

    pad = E_PAD - E_TOT
    pad_idx = jnp.arange(pad, dtype=jnp.int32) % N
    src = jnp.concatenate([src, pad_idx]).reshape(NW, NCH, CHUNK)
    dst = jnp.concatenate([dst, pad_idx]).reshape(NW, NCH, CHUNK)
    ew = jnp.concatenate([ew, jnp.zeros((pad,), jnp.float32)])
    ew = ew.reshape(NW, NCH, CHUNK)

    partial = _sc_scatter(xw, src, dst, ew)
    return _tc_combine(partial)


# speedup vs baseline: 1.0027x; 1.0027x over previous
import functools

import jax
import jax.numpy as jnp
from jax import lax
from jax.experimental import pallas as pl
from jax.experimental.pallas import tpu as pltpu
from jax.experimental.pallas import tpu_sc as plsc

N = 10000
D = 128
NS_SUP = 2
E_TOT = 2 * 320000
NC = 2
NSC = 16
NW = NC * NSC
CHUNK = 128
EB = 8
NCH = 160
NB = NCH // EB
E_PAD = NW * CHUNK * NCH
SHARE = 624
SHARE_SPLIT = (128, 128, 128, 128, 112)
REM_START = NSC * SHARE
REM = N - REM_START


def _mm_body(x_ref, w_ref, o_ref):
    o_ref[...] = jnp.dot(x_ref[...], w_ref[0],
                         preferred_element_type=jnp.float32)[None]


def _tc_matmul(x, W):
    BR = 2000
    out = pl.pallas_call(
        _mm_body,
        grid=(NS_SUP, N // BR),
        in_specs=[
            pl.BlockSpec((BR, D), lambda s, i: (i, 0)),
            pl.BlockSpec((1, D, D), lambda s, i: (s, 0, 0)),
        ],
        out_specs=pl.BlockSpec((1, BR, D), lambda s, i: (s, i, 0)),
        out_shape=jax.ShapeDtypeStruct((NS_SUP, N, D), jnp.float32),
    )(x, W)
    return out.reshape(NS_SUP * N, D)


def _add_body(p_ref, o_ref):
    o_ref[...] = p_ref[0] + p_ref[1]


def _tc_combine(partial):
    BR = 2000
    return pl.pallas_call(
        _add_body,
        grid=(N // BR,),
        in_specs=[pl.BlockSpec((NC, BR, D), lambda i: (0, i, 0))],
        out_specs=pl.BlockSpec((BR, D), lambda i: (i, 0)),
        out_shape=jax.ShapeDtypeStruct((N, D), jnp.float32),
    )(partial)


def _splat_lane(vec, lane):
    idx = jnp.full((16, 1), lane, jnp.int32)
    return lax.gather(
        vec, idx,
        lax.GatherDimensionNumbers(
            offset_dims=(), collapsed_slice_dims=(0,), start_index_map=(0,)),
        slice_sizes=(1,),
        mode=lax.GatherScatterMode.PROMISE_IN_BOUNDS)


_sc_mesh = plsc.VectorSubcoreMesh(
    core_axis_name="c", subcore_axis_name="s", num_cores=NC, num_subcores=NSC
)


@functools.partial(
    pl.kernel,
    out_type=jax.ShapeDtypeStruct((NC, N, D), jnp.float32),
    mesh=_sc_mesh,
    scratch_types=[
        pltpu.VMEM((2, EB, CHUNK), jnp.int32),
        pltpu.VMEM((2, EB, CHUNK), jnp.int32),
        pltpu.VMEM((2, EB, CHUNK), jnp.float32),
        pltpu.VMEM((CHUNK, D), jnp.float32),
        pltpu.VMEM((CHUNK, D), jnp.float32),
        pltpu.VMEM_SHARED((N, D), jnp.float32),
        pltpu.SemaphoreType.DMA,
        pltpu.SemaphoreType.DMA,
        pltpu.SemaphoreType.DMA,
        pltpu.SemaphoreType.DMA,
        pltpu.SemaphoreType.DMA,
    ],
)
def _sc_scatter(xw_hbm, src_hbm, dst_hbm, ew_hbm, out_hbm,
                src_v, dst_v, ew_v, rows0_v, rows1_v, acc,
                gsem_a0, gsem_a1, gsem_b0, gsem_b1, esem):
    rows_v = rows0_v
    H = CHUNK // 2

    def _gather_start(es, j, rows, s0, s1):
        pltpu.async_copy(xw_hbm.at[src_v.at[es, j, pl.ds(0, H)]],
                         rows.at[pl.ds(0, H)], s0)
        pltpu.async_copy(xw_hbm.at[src_v.at[es, j, pl.ds(H, H)]],
                         rows.at[pl.ds(H, H)], s1)

    def _gather_wait(es, j, rows, s0, s1):
        pltpu.make_async_copy(xw_hbm.at[src_v.at[es, j, pl.ds(0, H)]],
                              rows.at[pl.ds(0, H)], s0).wait()
        pltpu.make_async_copy(xw_hbm.at[src_v.at[es, j, pl.ds(H, H)]],
                              rows.at[pl.ds(H, H)], s1).wait()
    cid = lax.axis_index("c")
    sid = lax.axis_index("s")
    wid = cid * NSC + sid

    def _zero_body(i, carry):
        z = jnp.zeros((16,), jnp.float32)
        for g in range(8):
            rows_v[i, pl.ds(g * 16, 16)] = z
        return carry

    lax.fori_loop(0, CHUNK, _zero_body, 0)
    off = 0
    for ln in SHARE_SPLIT:
        pltpu.sync_copy(rows_v.at[pl.ds(0, ln)],
                        acc.at[pl.ds(sid * SHARE + off, ln)])
        off += ln

    @pl.when(sid == 0)
    def _zero_rem():
        pltpu.sync_copy(rows_v.at[pl.ds(0, REM)], acc.at[pl.ds(REM_START, REM)])

    plsc.subcore_barrier()

    def _scale(rows, es, j):
        def _group_body(gr, c2):
            wv = ew_v[es, j, pl.ds(gr * 16, 16)]
            for ln in range(16):
                w = _splat_lane(wv, ln)
                e = gr * 16 + ln
                for g in range(8):
                    rows[e, pl.ds(g * 16, 16)] = rows[e, pl.ds(g * 16, 16)] * w
            return c2

        lax.fori_loop(0, CHUNK // 16, _group_body, 0)

    def _stage_start(b, slot):
        bsl = pl.ds(b * EB, EB)
        pltpu.async_copy(src_hbm.at[wid, bsl], src_v.at[slot], esem)
        pltpu.async_copy(dst_hbm.at[wid, bsl], dst_v.at[slot], esem)
        pltpu.async_copy(ew_hbm.at[wid, bsl], ew_v.at[slot], esem)

    def _stage_wait(b, slot):
        bsl = pl.ds(b * EB, EB)
        pltpu.make_async_copy(src_hbm.at[wid, bsl], src_v.at[slot], esem).wait()
        pltpu.make_async_copy(dst_hbm.at[wid, bsl], dst_v.at[slot], esem).wait()
        pltpu.make_async_copy(ew_hbm.at[wid, bsl], ew_v.at[slot], esem).wait()

    _stage_start(0, 0)
    _stage_wait(0, 0)

    def _block_body(b, carry):
        es = b & 1
        @pl.when(b < NB - 1)
        def _stage_next():
            _stage_start(b + 1, 1 - es)

        _gather_start(es, 0, rows0_v, gsem_a0, gsem_a1)

        def _pair_body(p, c1):
            ja = 2 * p
            jb = ja + 1
            _gather_wait(es, ja, rows0_v, gsem_a0, gsem_a1)
            _gather_start(es, jb, rows1_v, gsem_b0, gsem_b1)
            _scale(rows0_v, es, ja)
            pltpu.sync_copy(rows0_v, acc.at[dst_v.at[es, ja]], add=True)

            _gather_wait(es, jb, rows1_v, gsem_b0, gsem_b1)

            @pl.when(p < EB // 2 - 1)
            def _prefetch_next():
                _gather_start(es, ja + 2, rows0_v, gsem_a0, gsem_a1)

            _scale(rows1_v, es, jb)
            pltpu.sync_copy(rows1_v, acc.at[dst_v.at[es, jb]], add=True)
            return c1

        lax.fori_loop(0, EB // 2, _pair_body, 0)

        @pl.when(b < NB - 1)
        def _stage_drain():
            _stage_wait(b + 1, 1 - es)

        return carry

    lax.fori_loop(0, NB, _block_body, 0)
    plsc.subcore_barrier()

    off = 0
    for ln in SHARE_SPLIT:
        sl = pl.ds(sid * SHARE + off, ln)
        pltpu.sync_copy(acc.at[sl], out_hbm.at[cid, sl])
        off += ln

    @pl.when(sid == 0)
    def _write_rem():
        sl = pl.ds(REM_START, REM)
        pltpu.sync_copy(acc.at[sl], out_hbm.at[cid, sl])


def kernel(x, edge_index_0, edge_weight_0, edge_index_1, edge_weight_1, W):
    xw = _tc_matmul(x, W)

    src = jnp.concatenate([
        edge_index_0[1].astype(jnp.int32),
        edge_index_1[1].astype(jnp.int32) + N,
    ])
    dst = jnp.concatenate([
        edge_index_0[0].astype(jnp.int32),
        edge_index_1[0].astype(jnp.int32),
    ])
    ew = jnp.concatenate([edge_weight_0, edge_weight_1])

    pad = E_PAD - E_TOT
    pad_idx = jnp.arange(pad, dtype=jnp.int32) % N
    src = jnp.concatenate([src, pad_idx]).reshape(NW, NCH, CHUNK)
    dst = jnp.concatenate([dst, pad_idx]).reshape(NW, NCH, CHUNK)
    ew = jnp.concatenate([ew, jnp.zeros((pad,), jnp.float32)])
    ew = ew.reshape(NW, NCH, CHUNK)

    partial = _sc_scatter(xw, src, dst, ew)
    return _tc_combine(partial)
